# Initial kernel scaffold; baseline (speedup 1.0000x reference)
import jax
import jax.numpy as jnp
from jax.experimental import pallas as pl


def kernel(x, gate_w, expert_bias, gate_up_weight, down_weight, shared_gate_up, shared_down):
    raise NotImplementedError("write your pallas kernel here")


# speedup vs baseline: 5.7959x; 5.7959x over previous
import functools

import jax
import jax.numpy as jnp
from jax.experimental import pallas as pl
from jax.experimental.pallas import tpu as pltpu

T = 1024
D = 768
E = 64
F = 128
BT = 16
P = 2048
G = P // BT
Z_COEF = 1e-3


def _plan_body(x_ref, gw_ref, bias_ref, sgu_ref, sd_ref,
               z_ref, g16_ref, slot_ref, be_ref, shared_ref):
    x = x_ref[...]
    gw = gw_ref[...]
    logits = jax.lax.dot_general(x, gw, (((1,), (1,)), ((), ())),
                                 preferred_element_type=jnp.float32)
    m = jnp.max(logits, axis=1, keepdims=True)
    lse = m + jnp.log(jnp.sum(jnp.exp(logits - m), axis=1, keepdims=True))
    z_ref[...] = (Z_COEF * jnp.mean(lse * lse)).reshape(1, 1)

    affinity = jax.nn.sigmoid(logits)
    scores = affinity + bias_ref[...]
    smax = jnp.max(scores, axis=1, keepdims=True)
    lane = jax.lax.broadcasted_iota(jnp.int32, (T, E), 1)
    sel = jnp.min(jnp.where(scores >= smax, lane, E), axis=1, keepdims=True)
    onehot_b = lane == sel
    onehot = onehot_b.astype(jnp.float32)
    aff_sel = jnp.sum(jnp.where(onehot_b, affinity, 0.0), axis=1, keepdims=True)
    gating = aff_sel / (aff_sel + 1e-9)
    g16_ref[...] = jnp.broadcast_to(gating, (T, 16))

    counts = jnp.sum(onehot, axis=0, keepdims=True)
    r_i = jax.lax.broadcasted_iota(jnp.int32, (T, T), 0)
    c_i = jax.lax.broadcasted_iota(jnp.int32, (T, T), 1)
    tril = (c_i <= r_i).astype(jnp.float32)
    incl = jax.lax.dot_general(tril, onehot, (((1,), (0,)), ((), ())),
                               preferred_element_type=jnp.float32)
    rank = jnp.sum(jnp.where(onehot_b, incl, 0.0), axis=1, keepdims=True) - 1.0
    nb = jnp.floor((counts + (BT - 1)) * (1.0 / BT))
    e_r = jax.lax.broadcasted_iota(jnp.int32, (E, E), 0)
    e_c = jax.lax.broadcasted_iota(jnp.int32, (E, E), 1)
    tril_excl = (e_r < e_c).astype(jnp.float32)
    bstart = jax.lax.dot_general(nb, tril_excl, (((1,), (0,)), ((), ())),
                                 preferred_element_type=jnp.float32)
    bsel = jnp.sum(jnp.where(onehot_b, jnp.broadcast_to(bstart, (T, E)), 0.0),
                   axis=1, keepdims=True)
    slot_ref[...] = (BT * bsel + rank).astype(jnp.int32)

    g_i = jax.lax.broadcasted_iota(jnp.int32, (G, E), 0)
    bstart_i = bstart.astype(jnp.int32)
    be_ref[...] = (jnp.sum((bstart_i <= g_i).astype(jnp.int32), axis=1,
                           keepdims=True) - 1)

    sc = jax.lax.dot_general(x, sgu_ref[...], (((1,), (1,)), ((), ())),
                             preferred_element_type=jnp.float32)
    sg = sc[:, :F]
    su = sc[:, F:]
    h = sg * jax.nn.sigmoid(sg) * su
    shared_ref[...] = jax.lax.dot_general(h, sd_ref[...], (((1,), (1,)), ((), ())),
                                          preferred_element_type=jnp.float32)


_plan_call = pl.pallas_call(
    _plan_body,
    out_shape=(
        jax.ShapeDtypeStruct((1, 1), jnp.float32),
        jax.ShapeDtypeStruct((T, 16), jnp.float32),
        jax.ShapeDtypeStruct((T, 1), jnp.int32),
        jax.ShapeDtypeStruct((G, 1), jnp.int32),
        jax.ShapeDtypeStruct((T, D), jnp.float32),
    ),
)


def _ffn_body(be_ref, xs_ref, gu_ref, dn_ref, gs_ref, out_ref):
    xb = xs_ref[...]
    gu = gu_ref[0]
    comb = jax.lax.dot_general(xb, gu, (((1,), (1,)), ((), ())),
                               preferred_element_type=jnp.float32)
    gate = comb[:, :F]
    up = comb[:, F:]
    h = gate * jax.nn.sigmoid(gate) * up
    dn = dn_ref[0]
    o = jax.lax.dot_general(h, dn, (((1,), (1,)), ((), ())),
                            preferred_element_type=jnp.float32)
    out_ref[...] = o * gs_ref[:, :1]


_ffn_call = pl.pallas_call(
    _ffn_body,
    grid_spec=pltpu.PrefetchScalarGridSpec(
        num_scalar_prefetch=1,
        grid=(G,),
        in_specs=[
            pl.BlockSpec((BT, D), lambda g, be: (g, 0)),
            pl.BlockSpec((1, 2 * F, D), lambda g, be: (be[g], 0, 0)),
            pl.BlockSpec((1, D, F), lambda g, be: (be[g], 0, 0)),
            pl.BlockSpec((BT, 16), lambda g, be: (g, 0)),
        ],
        out_specs=pl.BlockSpec((BT, D), lambda g, be: (g, 0)),
    ),
    out_shape=jax.ShapeDtypeStruct((P, D), jnp.float32),
)


def kernel(x, gate_w, expert_bias, gate_up_weight, down_weight,
           shared_gate_up, shared_down):
    z, g16, slot_col, be_col, shared = _plan_call(
        x, gate_w, expert_bias.reshape(1, E), shared_gate_up, shared_down)
    slot = slot_col.reshape(T)
    be = be_col.reshape(G)

    xs = jnp.zeros((P, D), jnp.float32).at[slot].set(x)
    gs = jnp.zeros((P, 16), jnp.float32).at[slot].set(g16)

    out_padded = _ffn_call(be, xs, gate_up_weight, down_weight, gs)

    routed = jnp.take(out_padded, slot, axis=0)
    out = routed + shared
    return out, z.reshape(())
